# Initial kernel scaffold; baseline (speedup 1.0000x reference)
import jax
import jax.numpy as jnp
from jax.experimental import pallas as pl


def kernel(x):
    raise NotImplementedError("write your pallas kernel here")


# speedup vs baseline: 5.0098x; 5.0098x over previous
import math

import jax
import jax.numpy as jnp
import numpy as np
from jax import lax
from jax.experimental import pallas as pl

_T = 2048
_PRED = 256
_K = 16
_NF = 1023
_NFP = 1024
_BC = 128

_mm = np.arange(1, _NF + 1, dtype=np.int64)
_tt = np.arange(_T, dtype=np.int64)
_ang = (2.0 * math.pi / _T) * ((_tt[:, None] * _mm[None, :]) % _T)
_CT = np.zeros((_T, _NFP), np.float32)
_ST = np.zeros((_T, _NFP), np.float32)
_CT[:, :_NF] = np.cos(_ang)
_ST[:, :_NF] = np.sin(_ang)


def _body(mag_ref, re_ref, im_ref, ct_ref, st_ref, o_ref):
    mag = mag_ref[...]
    rowid = lax.broadcasted_iota(jnp.int32, (_NFP, _BC), 0)
    mask = jnp.zeros((_NFP, _BC), jnp.bool_)
    m = mag
    for _ in range(_K):
        mx = jnp.max(m, axis=0, keepdims=True)
        ismax = m == mx
        first = jnp.min(jnp.where(ismax, rowid, _NFP), axis=0, keepdims=True)
        sel = rowid == first
        mask = jnp.logical_or(mask, sel)
        m = jnp.where(sel, jnp.float32(-1.0), m)
    scale = jnp.float32(2.0 / _T)
    pm = jnp.where(mask, re_ref[...], jnp.float32(0.0)) * scale
    qm = jnp.where(mask, im_ref[...], jnp.float32(0.0)) * scale
    dn = (((1,), (0,)), ((), ()))
    out = lax.dot_general(ct_ref[...], pm, dn,
                          precision=lax.Precision.HIGHEST,
                          preferred_element_type=jnp.float32)
    out = out - lax.dot_general(st_ref[...], qm, dn,
                                precision=lax.Precision.HIGHEST,
                                preferred_element_type=jnp.float32)
    o_ref[...] = out


def kernel(x):
    b, t, d = x.shape
    xf = jnp.fft.rfft(x, axis=1)[:, 1:-1]
    mag = jnp.abs(xf)
    re = jnp.real(xf)
    im = jnp.imag(xf)

    def to_cols(a):
        a = a.transpose(1, 0, 2).reshape(_NF, b * d)
        return jnp.pad(a, ((0, _NFP - _NF), (0, 0)))

    mag_c, re_c, im_c = to_cols(mag), to_cols(re), to_cols(im)
    cols = b * d
    grid = (cols // _BC,)
    out = pl.pallas_call(
        _body,
        grid=grid,
        in_specs=[
            pl.BlockSpec((_NFP, _BC), lambda j: (0, j)),
            pl.BlockSpec((_NFP, _BC), lambda j: (0, j)),
            pl.BlockSpec((_NFP, _BC), lambda j: (0, j)),
            pl.BlockSpec((_T, _NFP), lambda j: (0, 0)),
            pl.BlockSpec((_T, _NFP), lambda j: (0, 0)),
        ],
        out_specs=pl.BlockSpec((_T, _BC), lambda j: (0, j)),
        out_shape=jax.ShapeDtypeStruct((_T, cols), jnp.float32),
    )(mag_c, re_c, im_c, jnp.asarray(_CT), jnp.asarray(_ST))

    out = out.reshape(_T, b, d).transpose(1, 0, 2)
    return jnp.concatenate([out, out[:, :_PRED]], axis=1)
